# baseline (speedup 1.0000x reference)
import functools
import jax
import jax.numpy as jnp
from jax.experimental import pallas as pl
from jax.experimental.pallas import tpu as pltpu

N = 50000
E = 800000
D = 128
H = 2
DH = 32
M = 64
R = 50
B = 8
L = 2
NC = 10
CUTOFF = 6.0

_NBLK = 2000


def _embed_body(x_ref, pos_ref, w1_ref, w2_ref, b_ref, o_ref):
    x = x_ref[...]
    pos = pos_ref[...]
    o_ref[...] = x @ w1_ref[...] + pos @ w2_ref[...] + b_ref[...]


def _h0(x, pos, W_embed, b_embed, W_init, b_init):
    w1 = W_embed @ W_init[:5]
    w2 = W_init[5:8]
    b = (b_embed @ W_init[:5] + b_init)[None, :]
    grid = (N // _NBLK,)
    return pl.pallas_call(
        _embed_body,
        grid=grid,
        in_specs=[
            pl.BlockSpec((_NBLK, 1), lambda i: (i, 0)),
            pl.BlockSpec((_NBLK, 3), lambda i: (i, 0)),
            pl.BlockSpec((1, D), lambda i: (0, 0)),
            pl.BlockSpec((3, D), lambda i: (0, 0)),
            pl.BlockSpec((1, D), lambda i: (0, 0)),
        ],
        out_specs=pl.BlockSpec((_NBLK, D), lambda i: (i, 0)),
        out_shape=jax.ShapeDtypeStruct((N, D), jnp.float32),
    )(x, pos, w1, w2, b)


def _ln(x, g, b, eps=1e-5):
    mu = jnp.mean(x, axis=-1, keepdims=True)
    v = jnp.mean((x - mu) ** 2, axis=-1, keepdims=True)
    return (x - mu) / jnp.sqrt(v + eps) * g + b


def _segment_softmax(logits, seg, n):
    m = jax.ops.segment_max(logits, seg, num_segments=n)
    m = jnp.where(jnp.isfinite(m), m, 0.0)
    e = jnp.exp(logits - m[seg])
    s = jax.ops.segment_sum(e, seg, num_segments=n)
    return e / (s[seg] + 1e-16)


def kernel(x, pos, edge_index, batch, W_embed, b_embed, W_init, b_init, Wq, Wk, Wv, We, Wo, bo, Wfc, bfc, ln_g, ln_b, W_pool, b_pool, Wg, bg, lng, lnb, Wna1, bna1, Wna2, bna2, Wcp1, bcp1, Wcp2, bcp2):
    src = edge_index[0]
    dst = edge_index[1]
    h = _h0(x, pos, W_embed, b_embed, W_init, b_init)
    d = jnp.sqrt(jnp.sum((pos[src] - pos[dst]) ** 2, axis=1) + 1e-12)
    mu = jnp.linspace(0.0, CUTOFF, R)
    gamma = 1.0 / (mu[1] - mu[0]) ** 2
    rbf = jnp.exp(-gamma * (d[:, None] - mu[None, :]) ** 2)
    for l in range(L):
        q = (h @ Wq[l]).reshape(N, H, DH)
        kk = (h @ Wk[l]).reshape(N, H, DH)
        v = (h @ Wv[l]).reshape(N, H, DH)
        e = (rbf @ We[l]).reshape(E, H, DH)
        kj = kk[src] + e
        vj = v[src] + e
        logits = jnp.sum(q[dst] * kj, axis=-1) / jnp.sqrt(float(DH))
        alpha = _segment_softmax(logits, dst, N)
        msg = alpha[:, :, None] * vj
        agg = jax.ops.segment_sum(msg, dst, num_segments=N).reshape(N, M)
        h = h + agg @ Wo[l] + bo[l]
        h = _ln(h + jax.nn.gelu(h @ Wfc[l] + bfc[l]), ln_g[l], ln_b[l])
    ssum = jax.ops.segment_sum(h, batch, num_segments=B)
    cnt = jax.ops.segment_sum(jnp.ones((N,)), batch, num_segments=B)
    smean = ssum / (cnt[:, None] + 1e-8)
    smax = jax.ops.segment_max(h, batch, num_segments=B)
    smax = jnp.where(jnp.isfinite(smax), smax, 0.0)
    g = jnp.concatenate([ssum, smean, smax], axis=1) @ W_pool + b_pool
    for i in range(2):
        g = _ln(g + jax.nn.gelu(g @ Wg[i] + bg[i]), lng[i], lnb[i])
    enc = g
    na = jax.nn.gelu(enc @ Wna1 + bna1) @ Wna2 + bna2
    cp = jax.nn.gelu(enc @ Wcp1 + bcp1) @ Wcp2 + bcp2
    return (enc, na, cp)


# speedup vs baseline: 16.5300x; 16.5300x over previous
import functools
import jax
import jax.numpy as jnp
from jax import lax
from jax.experimental import pallas as pl
from jax.experimental.pallas import tpu as pltpu
from jax.experimental.pallas import tpu_sc as plsc

N = 50000
E = 800000
D = 128
H = 2
DH = 32
M = 64
R = 50
B = 8
L = 2
NC = 10
CUTOFF = 6.0

_NBLK = 2000
_EBLKTC = 4000
_EBLK = 128
_NW = 32
_WCHUNK = 25088
_WLAST = E - 31 * _WCHUNK
_EBLKA = 80
_WCHUNKA = 25040
_WLASTA = E - 31 * _WCHUNKA
_NPT = N // 16

_mesh = plsc.VectorSubcoreMesh(core_axis_name="c", subcore_axis_name="s")


def _wid():
    return lax.axis_index("c") * 16 + lax.axis_index("s")


def _nblk(wid):
    return jnp.where(wid == _NW - 1, _WLAST // _EBLK, _WCHUNK // _EBLK)


def _nblka(wid):
    return jnp.where(wid == _NW - 1, _WLASTA // _EBLKA, _WCHUNKA // _EBLKA)


def _sc_pos_body(src_h, dst_h, pos_h, psrc_h, pdst_h, idxs, idxd, rs, rd, sem):
    wid = _wid()
    base0 = wid * _WCHUNK
    nblk = _nblk(wid)

    def blk(i, carry):
        base = base0 + i * _EBLK
        pltpu.sync_copy(src_h.at[pl.ds(base, _EBLK)], idxs)
        pltpu.sync_copy(dst_h.at[pl.ds(base, _EBLK)], idxd)
        c1 = pltpu.async_copy(pos_h.at[idxs], rs, sem)
        c1.wait()
        c2 = pltpu.async_copy(pos_h.at[idxd], rd, sem)
        c2.wait()
        pltpu.sync_copy(rs, psrc_h.at[pl.ds(base, _EBLK)])
        pltpu.sync_copy(rd, pdst_h.at[pl.ds(base, _EBLK)])
        return carry

    lax.fori_loop(0, nblk, blk, 0)


@functools.partial(
    pl.kernel,
    out_type=[
        jax.ShapeDtypeStruct((E, 16), jnp.float32),
        jax.ShapeDtypeStruct((E, 16), jnp.float32),
    ],
    mesh=_mesh,
    compiler_params=pltpu.CompilerParams(use_tc_tiling_on_sc=False, needs_layout_passes=False),
    scratch_types=[
        pltpu.VMEM((_EBLK,), jnp.int32),
        pltpu.VMEM((_EBLK,), jnp.int32),
        pltpu.VMEM((_EBLK, 16), jnp.float32),
        pltpu.VMEM((_EBLK, 16), jnp.float32),
        pltpu.SemaphoreType.DMA,
    ],
)
def _sc_pos_gather(src_h, dst_h, pos_h, psrc_h, pdst_h, idxs, idxd, rs, rd, sem):
    _sc_pos_body(src_h, dst_h, pos_h, psrc_h, pdst_h, idxs, idxd, rs, rd, sem)


def _sc_pass_a_body(src_h, dst_h, q_h, k_h, va_h, ea_h, eb_h, z32_h,
                    p0_h, p1_h, u1_h,
                    srcb, dstb, qst, kst, east, ebst, vast, mst, p0st, p1st,
                    u1_sh, sem):
    c = lax.axis_index("c")
    s = lax.axis_index("s")
    wid = c * 16 + s
    zoff = s * _NPT
    pltpu.sync_copy(z32_h, u1_sh.at[pl.ds(zoff, _NPT)])
    plsc.subcore_barrier()

    base0 = wid * _WCHUNKA
    nblk = _nblka(wid)
    rscale = 1.0 / (float(DH) ** 0.5)

    def blk(i, carry):
        base = base0 + i * _EBLKA
        pltpu.sync_copy(src_h.at[pl.ds(base, _EBLKA)], srcb)
        pltpu.sync_copy(dst_h.at[pl.ds(base, _EBLKA)], dstb)
        cq = pltpu.async_copy(q_h.at[dstb], qst, sem)
        ck = pltpu.async_copy(k_h.at[srcb], kst, sem)
        cv = pltpu.async_copy(va_h.at[srcb], vast, sem)
        ce = pltpu.async_copy(ea_h.at[pl.ds(base, _EBLKA)], east, sem)
        cf = pltpu.async_copy(eb_h.at[pl.ds(base, _EBLKA)], ebst, sem)
        cq.wait()
        ck.wait()
        cv.wait()
        ce.wait()
        cf.wait()

        def grp(g, carry2):
            ids = lax.iota(jnp.int32, 16) + g * 16
            acc0 = jnp.zeros((16,), jnp.float32)
            acc1 = jnp.zeros((16,), jnp.float32)
            for dh in range(DH):
                colA = jnp.full((16,), dh, jnp.int32)
                colB = jnp.full((16,), dh + DH, jnp.int32)
                qv0 = plsc.load_gather(qst, [ids, colA])
                kv0 = plsc.load_gather(kst, [ids, colA])
                ev0 = plsc.load_gather(east, [ids, colA])
                acc0 = acc0 + qv0 * (kv0 + ev0)
                qv1 = plsc.load_gather(qst, [ids, colB])
                kv1 = plsc.load_gather(kst, [ids, colB])
                ev1 = plsc.load_gather(ebst, [ids, colA])
                acc1 = acc1 + qv1 * (kv1 + ev1)
            p0st[pl.ds(g * 16, 16)] = jnp.exp(acc0 * rscale)
            p1st[pl.ds(g * 16, 16)] = jnp.exp(acc1 * rscale)
            return carry2

        lax.fori_loop(0, _EBLKA // 16, grp, 0)

        def edge(e, carry3):
            p0v = plsc.load_gather(p0st, [jnp.full((16,), e, jnp.int32)])
            lo = vast[e, pl.ds(0, 16)] + east[e, pl.ds(0, 16)]
            hi = vast[e, pl.ds(16, 16)] + east[e, pl.ds(16, 16)]
            mst[e, pl.ds(0, 16)] = p0v * lo
            mst[e, pl.ds(16, 16)] = p0v * hi
            return carry3

        lax.fori_loop(0, _EBLKA, edge, 0)
        pltpu.sync_copy(mst, u1_sh.at[dstb], add=True)
        pltpu.sync_copy(p0st, p0_h.at[pl.ds(base, _EBLKA)])
        pltpu.sync_copy(p1st, p1_h.at[pl.ds(base, _EBLKA)])
        return carry

    lax.fori_loop(0, nblk, blk, 0)
    plsc.subcore_barrier()
    woff = s * _NPT
    pltpu.sync_copy(u1_sh.at[pl.ds(woff, _NPT)],
                    u1_h.at[pl.ds(c * N + woff, _NPT)])


@functools.partial(
    pl.kernel,
    out_type=[
        jax.ShapeDtypeStruct((E,), jnp.float32),
        jax.ShapeDtypeStruct((E,), jnp.float32),
        jax.ShapeDtypeStruct((2 * N, 32), jnp.float32),
    ],
    mesh=_mesh,
    compiler_params=pltpu.CompilerParams(use_tc_tiling_on_sc=False, needs_layout_passes=False, internal_scratch_in_bytes=1 << 20),
    scratch_types=[
        pltpu.VMEM((_EBLKA,), jnp.int32),
        pltpu.VMEM((_EBLKA,), jnp.int32),
        pltpu.VMEM((_EBLKA, 64), jnp.float32),
        pltpu.VMEM((_EBLKA, 64), jnp.float32),
        pltpu.VMEM((_EBLKA, 32), jnp.float32),
        pltpu.VMEM((_EBLKA, 32), jnp.float32),
        pltpu.VMEM((_EBLKA, 32), jnp.float32),
        pltpu.VMEM((_EBLKA, 32), jnp.float32),
        pltpu.VMEM((_EBLKA,), jnp.float32),
        pltpu.VMEM((_EBLKA,), jnp.float32),
        pltpu.VMEM_SHARED((N, 32), jnp.float32),
        pltpu.SemaphoreType.DMA,
    ],
)
def _sc_pass_a(src_h, dst_h, q_h, k_h, va_h, ea_h, eb_h, z32_h,
               p0_h, p1_h, u1_h,
               srcb, dstb, qst, kst, east, ebst, vast, mst, p0st, p1st,
               u1_sh, sem):
    _sc_pass_a_body(src_h, dst_h, q_h, k_h, va_h, ea_h, eb_h, z32_h,
                    p0_h, p1_h, u1_h,
                    srcb, dstb, qst, kst, east, ebst, vast, mst, p0st, p1st,
                    u1_sh, sem)


def _sc_pass_b_body(src_h, dst_h, vb_h, eb_h, p1_h, z32_h, u2_h,
                    srcb, dstb, vbst, ebst, p1st, mst, u2_sh, sem):
    c = lax.axis_index("c")
    s = lax.axis_index("s")
    wid = c * 16 + s
    zoff = s * _NPT
    pltpu.sync_copy(z32_h, u2_sh.at[pl.ds(zoff, _NPT)])
    plsc.subcore_barrier()

    base0 = wid * _WCHUNK
    nblk = _nblk(wid)

    def blk(i, carry):
        base = base0 + i * _EBLK
        pltpu.sync_copy(src_h.at[pl.ds(base, _EBLK)], srcb)
        pltpu.sync_copy(dst_h.at[pl.ds(base, _EBLK)], dstb)
        pltpu.sync_copy(p1_h.at[pl.ds(base, _EBLK)], p1st)
        cv = pltpu.async_copy(vb_h.at[srcb], vbst, sem)
        ce = pltpu.async_copy(eb_h.at[pl.ds(base, _EBLK)], ebst, sem)
        cv.wait()
        ce.wait()

        def edge(e, carry3):
            p1v = plsc.load_gather(p1st, [jnp.full((16,), e, jnp.int32)])
            lo = vbst[e, pl.ds(0, 16)] + ebst[e, pl.ds(0, 16)]
            hi = vbst[e, pl.ds(16, 16)] + ebst[e, pl.ds(16, 16)]
            mst[e, pl.ds(0, 16)] = p1v * lo
            mst[e, pl.ds(16, 16)] = p1v * hi
            return carry3

        lax.fori_loop(0, _EBLK, edge, 0)
        pltpu.sync_copy(mst, u2_sh.at[dstb], add=True)
        return carry

    lax.fori_loop(0, nblk, blk, 0)
    plsc.subcore_barrier()
    woff = s * _NPT
    pltpu.sync_copy(u2_sh.at[pl.ds(woff, _NPT)],
                    u2_h.at[pl.ds(c * N + woff, _NPT)])


@functools.partial(
    pl.kernel,
    out_type=[jax.ShapeDtypeStruct((2 * N, 32), jnp.float32)],
    mesh=_mesh,
    compiler_params=pltpu.CompilerParams(use_tc_tiling_on_sc=False, needs_layout_passes=False),
    scratch_types=[
        pltpu.VMEM((_EBLK,), jnp.int32),
        pltpu.VMEM((_EBLK,), jnp.int32),
        pltpu.VMEM((_EBLK, 32), jnp.float32),
        pltpu.VMEM((_EBLK, 32), jnp.float32),
        pltpu.VMEM((_EBLK,), jnp.float32),
        pltpu.VMEM((_EBLK, 32), jnp.float32),
        pltpu.VMEM_SHARED((N, 32), jnp.float32),
        pltpu.SemaphoreType.DMA,
    ],
)
def _sc_pass_b(src_h, dst_h, vb_h, eb_h, p1_h, z32_h, u2_h,
               srcb, dstb, vbst, ebst, p1st, mst, u2_sh, sem):
    _sc_pass_b_body(src_h, dst_h, vb_h, eb_h, p1_h, z32_h, u2_h,
                    srcb, dstb, vbst, ebst, p1st, mst, u2_sh, sem)


def _sc_pass_c_body(dst_h, p0_h, p1_h, z16_h, den_h,
                    dstb, p0st, p1st, pst, den_sh, sem):
    c = lax.axis_index("c")
    s = lax.axis_index("s")
    wid = c * 16 + s
    zoff = s * _NPT
    pltpu.sync_copy(z16_h, den_sh.at[pl.ds(zoff, _NPT)])
    plsc.subcore_barrier()

    zero16 = jnp.zeros((16,), jnp.float32)

    def zrow(e, carry):
        pst[e, pl.ds(0, 16)] = zero16
        return carry

    lax.fori_loop(0, _EBLK, zrow, 0)

    base0 = wid * _WCHUNK
    nblk = _nblk(wid)
    col0 = jnp.zeros((16,), jnp.int32)
    col1 = jnp.ones((16,), jnp.int32)

    def blk(i, carry):
        base = base0 + i * _EBLK
        pltpu.sync_copy(dst_h.at[pl.ds(base, _EBLK)], dstb)
        pltpu.sync_copy(p0_h.at[pl.ds(base, _EBLK)], p0st)
        pltpu.sync_copy(p1_h.at[pl.ds(base, _EBLK)], p1st)

        def grp(g, carry2):
            ids = lax.iota(jnp.int32, 16) + g * 16
            p0v = p0st[pl.ds(g * 16, 16)]
            p1v = p1st[pl.ds(g * 16, 16)]
            plsc.store_scatter(pst, [ids, col0], p0v)
            plsc.store_scatter(pst, [ids, col1], p1v)
            return carry2

        lax.fori_loop(0, _EBLK // 16, grp, 0)
        pltpu.sync_copy(pst, den_sh.at[dstb], add=True)
        return carry

    lax.fori_loop(0, nblk, blk, 0)
    plsc.subcore_barrier()
    woff = s * _NPT
    pltpu.sync_copy(den_sh.at[pl.ds(woff, _NPT)],
                    den_h.at[pl.ds(c * N + woff, _NPT)])


@functools.partial(
    pl.kernel,
    out_type=[jax.ShapeDtypeStruct((2 * N, 16), jnp.float32)],
    mesh=_mesh,
    compiler_params=pltpu.CompilerParams(use_tc_tiling_on_sc=False, needs_layout_passes=False),
    scratch_types=[
        pltpu.VMEM((_EBLK,), jnp.int32),
        pltpu.VMEM((_EBLK,), jnp.float32),
        pltpu.VMEM((_EBLK,), jnp.float32),
        pltpu.VMEM((_EBLK, 16), jnp.float32),
        pltpu.VMEM_SHARED((N, 16), jnp.float32),
        pltpu.SemaphoreType.DMA,
    ],
)
def _sc_pass_c(dst_h, p0_h, p1_h, z16_h, den_h,
               dstb, p0st, p1st, pst, den_sh, sem):
    _sc_pass_c_body(dst_h, p0_h, p1_h, z16_h, den_h,
                    dstb, p0st, p1st, pst, den_sh, sem)


def _mm(a, b):
    return jnp.matmul(a, b, precision=lax.Precision.HIGHEST)


def _gelu(x):
    c = 0.7978845608028654
    u = c * (x + 0.044715 * x * x * x)
    t = jnp.exp(-2.0 * jnp.abs(u))
    th = (1.0 - t) / (1.0 + t)
    th = jnp.where(u >= 0, th, -th)
    return 0.5 * x * (1.0 + th)


def _ln_in(x, g, b, eps=1e-5):
    mu = jnp.mean(x, axis=-1, keepdims=True)
    v = jnp.mean((x - mu) ** 2, axis=-1, keepdims=True)
    return (x - mu) / jnp.sqrt(v + eps) * g + b


def _h0_body(x_ref, pos_ref, we_ref, be_ref, wi_ref, bi_ref, o_ref):
    xe = _mm(x_ref[...], we_ref[...]) + be_ref[...]
    hcat = jnp.concatenate([xe, pos_ref[...]], axis=1)
    o_ref[...] = _mm(hcat, wi_ref[...]) + bi_ref[...]


def _h0(x, pos, W_embed, b_embed, W_init, b_init):
    return pl.pallas_call(
        _h0_body,
        grid=(N // _NBLK,),
        in_specs=[
            pl.BlockSpec((_NBLK, 1), lambda i: (i, 0)),
            pl.BlockSpec((_NBLK, 3), lambda i: (i, 0)),
            pl.BlockSpec((1, 5), lambda i: (0, 0)),
            pl.BlockSpec((1, 5), lambda i: (0, 0)),
            pl.BlockSpec((8, D), lambda i: (0, 0)),
            pl.BlockSpec((1, D), lambda i: (0, 0)),
        ],
        out_specs=pl.BlockSpec((_NBLK, D), lambda i: (i, 0)),
        out_shape=jax.ShapeDtypeStruct((N, D), jnp.float32),
    )(x, pos, W_embed, b_embed[None, :], W_init, b_init[None, :])


def _qkv_body(h_ref, wq_ref, wk_ref, wv_ref, q_ref, k_ref, va_ref, vb_ref):
    h = h_ref[...]
    q_ref[...] = _mm(h, wq_ref[...])
    k_ref[...] = _mm(h, wk_ref[...])
    v = _mm(h, wv_ref[...])
    va_ref[...] = v[:, :DH]
    vb_ref[...] = v[:, DH:]


def _tc_qkv(h, Wq, Wk, Wv):
    return pl.pallas_call(
        _qkv_body,
        grid=(N // _NBLK,),
        in_specs=[
            pl.BlockSpec((_NBLK, D), lambda i: (i, 0)),
            pl.BlockSpec((D, M), lambda i: (0, 0)),
            pl.BlockSpec((D, M), lambda i: (0, 0)),
            pl.BlockSpec((D, M), lambda i: (0, 0)),
        ],
        out_specs=[
            pl.BlockSpec((_NBLK, M), lambda i: (i, 0)),
            pl.BlockSpec((_NBLK, M), lambda i: (i, 0)),
            pl.BlockSpec((_NBLK, DH), lambda i: (i, 0)),
            pl.BlockSpec((_NBLK, DH), lambda i: (i, 0)),
        ],
        out_shape=[
            jax.ShapeDtypeStruct((N, M), jnp.float32),
            jax.ShapeDtypeStruct((N, M), jnp.float32),
            jax.ShapeDtypeStruct((N, DH), jnp.float32),
            jax.ShapeDtypeStruct((N, DH), jnp.float32),
        ],
    )(h, Wq, Wk, Wv)


def _efeat_body(ps_ref, pd_ref, wa0_ref, wb0_ref, wa1_ref, wb1_ref,
                ea0_ref, eb0_ref, ea1_ref, eb1_ref):
    diff = ps_ref[...] - pd_ref[...]
    d2 = jnp.sum(diff * diff, axis=1, keepdims=True)
    d = jnp.sqrt(d2 + 1e-12)
    step = CUTOFF / (R - 1)
    gamma = 1.0 / (step * step)
    mu = lax.broadcasted_iota(jnp.int32, (1, R), 1).astype(jnp.float32) * step
    t = d - mu
    rbf = jnp.exp(-gamma * t * t)
    ea0_ref[...] = _mm(rbf, wa0_ref[...])
    eb0_ref[...] = _mm(rbf, wb0_ref[...])
    ea1_ref[...] = _mm(rbf, wa1_ref[...])
    eb1_ref[...] = _mm(rbf, wb1_ref[...])


def _tc_efeat(psrc, pdst, We):
    espec = pl.BlockSpec((_EBLKTC, DH), lambda i: (i, 0))
    eshape = jax.ShapeDtypeStruct((E, DH), jnp.float32)
    return pl.pallas_call(
        _efeat_body,
        grid=(E // _EBLKTC,),
        in_specs=[
            pl.BlockSpec((_EBLKTC, 16), lambda i: (i, 0)),
            pl.BlockSpec((_EBLKTC, 16), lambda i: (i, 0)),
            pl.BlockSpec((R, DH), lambda i: (0, 0)),
            pl.BlockSpec((R, DH), lambda i: (0, 0)),
            pl.BlockSpec((R, DH), lambda i: (0, 0)),
            pl.BlockSpec((R, DH), lambda i: (0, 0)),
        ],
        out_specs=[espec, espec, espec, espec],
        out_shape=[eshape, eshape, eshape, eshape],
    )(psrc, pdst, We[0][:, :DH], We[0][:, DH:], We[1][:, :DH], We[1][:, DH:])


def _node_update_body(h_ref, u1a_ref, u1b_ref, u2a_ref, u2b_ref,
                      da_ref, db_ref, woa_ref, wob_ref, bo_ref,
                      wfc_ref, bfc_ref, g_ref, b_ref, o_ref):
    u1 = u1a_ref[...] + u1b_ref[...]
    u2 = u2a_ref[...] + u2b_ref[...]
    den = da_ref[...] + db_ref[...]
    aggA = u1 / (den[:, 0:1] + 1e-16)
    aggB = u2 / (den[:, 1:2] + 1e-16)
    h = h_ref[...] + _mm(aggA, woa_ref[...]) + _mm(aggB, wob_ref[...]) + bo_ref[...]
    ff = jax.nn.gelu(_mm(h, wfc_ref[...]) + bfc_ref[...])
    o_ref[...] = _ln_in(h + ff, g_ref[...], b_ref[...])


def _tc_node_update(h, u1, u2, den, Wo, bo, Wfc, bfc, ln_g, ln_b):
    nb = N // _NBLK
    return pl.pallas_call(
        _node_update_body,
        grid=(nb,),
        in_specs=[
            pl.BlockSpec((_NBLK, D), lambda i: (i, 0)),
            pl.BlockSpec((_NBLK, 32), lambda i: (i, 0)),
            pl.BlockSpec((_NBLK, 32), lambda i, _nb=nb: (i + _nb, 0)),
            pl.BlockSpec((_NBLK, 32), lambda i: (i, 0)),
            pl.BlockSpec((_NBLK, 32), lambda i, _nb=nb: (i + _nb, 0)),
            pl.BlockSpec((_NBLK, 16), lambda i: (i, 0)),
            pl.BlockSpec((_NBLK, 16), lambda i, _nb=nb: (i + _nb, 0)),
            pl.BlockSpec((DH, D), lambda i: (0, 0)),
            pl.BlockSpec((DH, D), lambda i: (0, 0)),
            pl.BlockSpec((1, D), lambda i: (0, 0)),
            pl.BlockSpec((D, D), lambda i: (0, 0)),
            pl.BlockSpec((1, D), lambda i: (0, 0)),
            pl.BlockSpec((1, D), lambda i: (0, 0)),
            pl.BlockSpec((1, D), lambda i: (0, 0)),
        ],
        out_specs=pl.BlockSpec((_NBLK, D), lambda i: (i, 0)),
        out_shape=jax.ShapeDtypeStruct((N, D), jnp.float32),
    )(h, u1, u1, u2, u2, den, den, Wo[:DH], Wo[DH:], bo[None, :],
      Wfc, bfc[None, :], ln_g[None, :], ln_b[None, :])


def _pool_body(batch_ref, h_ref, wp1_ref, wp2_ref, wp3_ref, bp_ref,
               wg_ref, bg_ref, lng_ref, lnb_ref,
               wna1_ref, bna1_ref, wna2_ref, bna2_ref,
               wcp1_ref, bcp1_ref, wcp2_ref, bcp2_ref,
               enc_ref, na_ref, cp_ref, ssum_scr, cnt_scr, smax_scr):
    i = pl.program_id(0)

    @pl.when(i == 0)
    def _():
        ssum_scr[...] = jnp.zeros_like(ssum_scr)
        cnt_scr[...] = jnp.zeros_like(cnt_scr)
        smax_scr[...] = jnp.full_like(smax_scr, -jnp.inf)

    h = h_ref[...]
    bcol = batch_ref[0].reshape(_NBLK, 1)
    onehot = (bcol == lax.broadcasted_iota(jnp.int32, (1, B), 1))
    onef = onehot.astype(jnp.float32)
    ssum_scr[...] += lax.dot_general(onef, h, (((0,), (0,)), ((), ())), precision=lax.Precision.HIGHEST)
    cnt_scr[...] += jnp.sum(onef, axis=0, keepdims=True)
    for b8 in range(B):
        mask = bcol == b8
        hm = jnp.where(mask, h, -jnp.inf)
        mrow = jnp.max(hm, axis=0, keepdims=True)
        smax_scr[pl.ds(b8, 1), :] = jnp.maximum(smax_scr[pl.ds(b8, 1), :], mrow)

    @pl.when(i == (N // _NBLK) - 1)
    def _():
        ssum = ssum_scr[...]
        cnt = cnt_scr[...].reshape(B, 1)
        smean = ssum / (cnt + 1e-8)
        smax = smax_scr[...]
        smax = jnp.where(jnp.isfinite(smax), smax, 0.0)
        g = (_mm(ssum, wp1_ref[...]) + _mm(smean, wp2_ref[...]) + _mm(smax, wp3_ref[...])
             + bp_ref[...])
        for i2 in range(2):
            gg = jax.nn.gelu(_mm(g, wg_ref[i2]) + bg_ref[pl.ds(i2, 1), :])
            g = _ln_in(g + gg, lng_ref[pl.ds(i2, 1), :], lnb_ref[pl.ds(i2, 1), :])
        enc_ref[...] = g
        na_ref[...] = _mm(jax.nn.gelu(_mm(g, wna1_ref[...]) + bna1_ref[...]), wna2_ref[...]) + bna2_ref[...]
        cp_ref[...] = _mm(jax.nn.gelu(_mm(g, wcp1_ref[...]) + bcp1_ref[...]), wcp2_ref[...]) + bcp2_ref[...]


def _tc_pool(batch3, h, W_pool, b_pool, Wg, bg, lng, lnb,
             Wna1, bna1, Wna2, bna2, Wcp1, bcp1, Wcp2, bcp2):
    full = lambda shape: pl.BlockSpec(shape, lambda i: tuple(0 for _ in shape))
    return pl.pallas_call(
        _pool_body,
        grid=(N // _NBLK,),
        in_specs=[
            pl.BlockSpec((1, 1, _NBLK), lambda i: (i, 0, 0)),
            pl.BlockSpec((_NBLK, D), lambda i: (i, 0)),
            full((D, D)), full((D, D)), full((D, D)), full((1, D)),
            full((2, D, D)), full((2, D)), full((2, D)), full((2, D)),
            full((D, 32)), full((1, 32)), full((32, 1)), full((1, 1)),
            full((D, 32)), full((1, 32)), full((32, NC)), full((1, NC)),
        ],
        out_specs=[
            pl.BlockSpec((B, D), lambda i: (0, 0)),
            pl.BlockSpec((B, 1), lambda i: (0, 0)),
            pl.BlockSpec((B, NC), lambda i: (0, 0)),
        ],
        out_shape=[
            jax.ShapeDtypeStruct((B, D), jnp.float32),
            jax.ShapeDtypeStruct((B, 1), jnp.float32),
            jax.ShapeDtypeStruct((B, NC), jnp.float32),
        ],
        scratch_shapes=[
            pltpu.VMEM((B, D), jnp.float32),
            pltpu.VMEM((1, B), jnp.float32),
            pltpu.VMEM((B, D), jnp.float32),
        ],
    )(batch3, h, W_pool[:D], W_pool[D:2 * D], W_pool[2 * D:], b_pool[None, :],
      Wg, bg, lng, lnb,
      Wna1, bna1[None, :], Wna2, bna2[None, :],
      Wcp1, bcp1[None, :], Wcp2, bcp2[None, :])


def _pool_tail_body(ssum_ref, smean_ref, smax_ref, wp1_ref, wp2_ref, wp3_ref,
                    bp_ref, wg_ref, bg_ref, lng_ref, lnb_ref,
                    wna1_ref, bna1_ref, wna2_ref, bna2_ref,
                    wcp1_ref, bcp1_ref, wcp2_ref, bcp2_ref,
                    enc_ref, na_ref, cp_ref):
    g = (_mm(ssum_ref[...], wp1_ref[...]) + _mm(smean_ref[...], wp2_ref[...])
         + _mm(smax_ref[...], wp3_ref[...]) + bp_ref[...])
    for i2 in range(2):
        gg = _gelu(_mm(g, wg_ref[i2]) + bg_ref[pl.ds(i2, 1), :])
        g = _ln_in(g + gg, lng_ref[pl.ds(i2, 1), :], lnb_ref[pl.ds(i2, 1), :])
    enc_ref[...] = g
    na_ref[...] = _mm(_gelu(_mm(g, wna1_ref[...]) + bna1_ref[...]), wna2_ref[...]) + bna2_ref[...]
    cp_ref[...] = _mm(_gelu(_mm(g, wcp1_ref[...]) + bcp1_ref[...]), wcp2_ref[...]) + bcp2_ref[...]


def _tc_pool_tail(ssum, smean, smax, W_pool, b_pool, Wg, bg, lng, lnb,
                  Wna1, bna1, Wna2, bna2, Wcp1, bcp1, Wcp2, bcp2):
    full = lambda shape: pl.BlockSpec(shape, lambda: tuple(0 for _ in shape))
    return pl.pallas_call(
        _pool_tail_body,
        in_specs=[
            full((B, D)), full((B, D)), full((B, D)),
            full((D, D)), full((D, D)), full((D, D)), full((1, D)),
            full((2, D, D)), full((2, D)), full((2, D)), full((2, D)),
            full((D, 32)), full((1, 32)), full((32, 1)), full((1, 1)),
            full((D, 32)), full((1, 32)), full((32, NC)), full((1, NC)),
        ],
        out_specs=[
            full((B, D)), full((B, 1)), full((B, NC)),
        ],
        out_shape=[
            jax.ShapeDtypeStruct((B, D), jnp.float32),
            jax.ShapeDtypeStruct((B, 1), jnp.float32),
            jax.ShapeDtypeStruct((B, NC), jnp.float32),
        ],
    )(ssum, smean, smax,
      W_pool[:D], W_pool[D:2 * D], W_pool[2 * D:], b_pool[None, :],
      Wg, bg, lng, lnb,
      Wna1, bna1[None, :], Wna2, bna2[None, :],
      Wcp1, bcp1[None, :], Wcp2, bcp2[None, :])


def kernel(x, pos, edge_index, batch, W_embed, b_embed, W_init, b_init, Wq, Wk, Wv, We, Wo, bo, Wfc, bfc, ln_g, ln_b, W_pool, b_pool, Wg, bg, lng, lnb, Wna1, bna1, Wna2, bna2, Wcp1, bcp1, Wcp2, bcp2):
    src = edge_index[0].astype(jnp.int32)
    dst = edge_index[1].astype(jnp.int32)
    pos16 = jnp.pad(pos, ((0, 0), (0, 13)))
    z32 = jnp.zeros((_NPT, 32), jnp.float32)
    z16 = jnp.zeros((_NPT, 16), jnp.float32)

    psrc, pdst = _sc_pos_gather(src, dst, pos16)
    h = _h0(x, pos, W_embed, b_embed, W_init, b_init)
    ea0, eb0, ea1, eb1 = _tc_efeat(psrc, pdst, We)
    eas = (ea0, ea1)
    ebs = (eb0, eb1)

    for l in range(L):
        q, k, va, vb = _tc_qkv(h, Wq[l], Wk[l], Wv[l])
        p0, p1, u1 = _sc_pass_a(src, dst, q, k, va, eas[l], ebs[l], z32)
        u2 = _sc_pass_b(src, dst, vb, ebs[l], p1, z32)
        if isinstance(u2, (list, tuple)):
            u2 = u2[0]
        den = _sc_pass_c(dst, p0, p1, z16)
        if isinstance(den, (list, tuple)):
            den = den[0]
        h = _tc_node_update(h, u1, u2, den, Wo[l], bo[l], Wfc[l], bfc[l],
                            ln_g[l], ln_b[l])

    batch3 = batch.astype(jnp.int32).reshape(N // _NBLK, 1, _NBLK)
    enc, na, cp = _tc_pool(batch3, h, W_pool, b_pool, Wg, bg, lng, lnb,
                           Wna1, bna1, Wna2, bna2, Wcp1, bcp1, Wcp2, bcp2)
    return (enc, na, cp)
